# baseline (speedup 1.0000x reference)
import functools

import jax
import jax.numpy as jnp
from jax import lax
from jax.experimental import pallas as pl
from jax.experimental.pallas import tpu as pltpu
from jax.experimental.pallas import tpu_sc as plsc

T = 2048
HIDDEN = 1024
FF = 512
E = 8
TOPK = 2
NPAIR = T * TOPK
BLK = 256
RPAD = NPAIR + E * BLK
NT = RPAD // BLK
NT1 = NT + 1
NW = 32
PW = NPAIR // NW
CW = 32
NC = PW // CW


def _router_body(x_ref, gate_ref, pos_ref, w_ref, te_ref):
    x = x_ref[...]
    gw = gate_ref[...]
    logits = lax.dot_general(
        x, gw, (((1,), (1,)), ((), ())), preferred_element_type=jnp.float32
    )
    lane = lax.broadcasted_iota(jnp.int32, (T, E), 1)
    m1 = jnp.max(logits, axis=1, keepdims=True)
    i1 = jnp.min(jnp.where(logits == m1, lane, E), axis=1, keepdims=True)
    oh1 = (lane == i1).astype(jnp.float32)
    logb = jnp.where(lane == i1, -jnp.inf, logits)
    m2 = jnp.max(logb, axis=1, keepdims=True)
    i2 = jnp.min(jnp.where(logb == m2, lane, E), axis=1, keepdims=True)
    oh2 = (lane == i2).astype(jnp.float32)
    e21 = jnp.exp(m2 - m1)
    w0 = 1.0 / (1.0 + e21)
    w1 = 1.0 - w0

    BS = 512
    r = lax.broadcasted_iota(jnp.int32, (BS, BS), 0)
    c = lax.broadcasted_iota(jnp.int32, (BS, BS), 1)
    ls = (c < r).astype(jnp.bfloat16)

    def excl_cumsum(oh):
        base = jnp.zeros((1, E), jnp.float32)
        parts = []
        for b in range(T // BS):
            sb = oh[b * BS:(b + 1) * BS]
            xb = lax.dot_general(
                ls, sb.astype(jnp.bfloat16), (((1,), (0,)), ((), ())),
                preferred_element_type=jnp.float32,
            ) + base
            parts.append(xb)
            base = base + jnp.sum(sb, axis=0, keepdims=True)
        return jnp.concatenate(parts, axis=0), base

    x1, c1 = excl_cumsum(oh1)
    x2, c2 = excl_cumsum(oh2)
    counts = c1 + c2
    padded = jnp.floor((counts + (BLK - 1)) * (1.0 / BLK)) * BLK
    acc = jnp.zeros((1, 1), jnp.float32)
    offs = []
    for e in range(E):
        offs.append(acc)
        acc = acc + padded[:, e:e + 1]
    pad_off = jnp.concatenate(offs, axis=1)
    total = acc

    rank1 = jnp.sum(x1 * oh1, axis=1, keepdims=True)
    rank2 = jnp.sum((c1 + x2) * oh2, axis=1, keepdims=True)
    off1 = jnp.sum(pad_off * oh1, axis=1, keepdims=True)
    off2 = jnp.sum(pad_off * oh2, axis=1, keepdims=True)
    pos1 = off1 + rank1
    pos2 = off2 + rank2
    pos_ref[...] = jnp.concatenate(
        [pos1, pos2], axis=0).astype(jnp.int32).reshape(NPAIR)
    w_ref[...] = jnp.concatenate([w0, w1], axis=0)

    jj = (lax.broadcasted_iota(jnp.int32, (NT1, E), 0) * BLK).astype(
        jnp.float32)
    te = jnp.sum((pad_off <= jj).astype(jnp.int32), axis=1, keepdims=True) - 1
    live = jj[:, 0:1] < total
    e8 = lax.broadcasted_iota(jnp.int32, (1, E), 1)
    last_e = jnp.max(jnp.where(padded > 0, e8, 0), axis=1, keepdims=True)
    n_live = (total * (1.0 / BLK)).astype(jnp.int32)
    tef = jnp.where(live, te, last_e)
    is_sent = lax.broadcasted_iota(jnp.int32, (NT1, 1), 0) == NT
    te_ref[...] = jnp.where(is_sent, n_live, tef).reshape(NT1)


def _wcast_body(wg_ref, wu_ref, wd_ref, wgb_ref, wub_ref, wdb_ref):
    wgb_ref[...] = wg_ref[...].astype(jnp.bfloat16)
    wub_ref[...] = wu_ref[...].astype(jnp.bfloat16)
    wdb_ref[...] = wd_ref[...].astype(jnp.bfloat16)


def _ffn_body(s_ref, xs_ref, wg_ref, wu_ref, wd_ref, y_ref):
    j = pl.program_id(0)
    e = s_ref[j]

    @pl.when(j < s_ref[NT])
    def _():
        xb = xs_ref[...].astype(jnp.bfloat16)
        g = lax.dot_general(
            xb, wg_ref[e], (((1,), (0,)), ((), ())),
            preferred_element_type=jnp.float32,
        )
        u = lax.dot_general(
            xb, wu_ref[e], (((1,), (0,)), ((), ())),
            preferred_element_type=jnp.float32,
        )
        h = (g * lax.logistic(g) * u).astype(jnp.bfloat16)
        y_ref[...] = lax.dot_general(
            h, wd_ref[e], (((1,), (0,)), ((), ())),
            preferred_element_type=jnp.float32,
        )


def _cadd_body(ya_ref, yb_ref, wa_ref, wb_ref, o_ref):
    o_ref[...] = ya_ref[...] * wa_ref[...] + yb_ref[...] * wb_ref[...]


@functools.cache
def _vector_mesh():
    return plsc.VectorSubcoreMesh(
        core_axis_name="c", subcore_axis_name="s", num_cores=2, num_subcores=16
    )


def _sc_dispatch(x, ppos):
    @functools.partial(
        pl.kernel,
        mesh=_vector_mesh(),
        out_type=jax.ShapeDtypeStruct((RPAD, HIDDEN), jnp.float32),
        scratch_types=[
            pltpu.VMEM((CW,), jnp.int32),
            pltpu.VMEM((CW,), jnp.int32),
            pltpu.VMEM((CW,), jnp.int32),
            pltpu.VMEM((CW,), jnp.int32),
            pltpu.VMEM((CW, HIDDEN), jnp.float32),
            pltpu.VMEM((CW, HIDDEN), jnp.float32),
            pltpu.SemaphoreType.DMA,
            pltpu.SemaphoreType.DMA,
            pltpu.SemaphoreType.DMA,
            pltpu.SemaphoreType.DMA,
        ],
    )
    def k(x_hbm, pos_hbm, xs_hbm, p0, p1, p2, p3, r0, r1,
          gs0, gs1, ss0, ss1):
        wid = lax.axis_index("s") * 2 + lax.axis_index("c")
        pbase = wid * PW
        tbase = pbase % T
        pv = (p0, p1, p2, p3)
        for ci in range(NC):
            pltpu.sync_copy(pos_hbm.at[pl.ds(pbase + ci * CW, CW)], pv[ci])
        rb = (r0, r1)
        gs = (gs0, gs1)
        ss = (ss0, ss1)
        g = [None] * NC
        s = [None] * NC
        for ci in range(2):
            g[ci] = pltpu.async_copy(
                x_hbm.at[pl.ds(tbase + ci * CW, CW)], rb[ci], gs[ci])
        for ci in range(NC):
            b = ci % 2
            g[ci].wait()
            s[ci] = pltpu.async_copy(rb[b], xs_hbm.at[pv[ci]], ss[b])
            if ci + 2 < NC:
                s[ci].wait()
                g[ci + 2] = pltpu.async_copy(
                    x_hbm.at[pl.ds(tbase + (ci + 2) * CW, CW)], rb[b], gs[b])
        s[NC - 2].wait()
        s[NC - 1].wait()

    return k(x, ppos)


def _sc_combine(ppos, y):
    @functools.partial(
        pl.kernel,
        mesh=_vector_mesh(),
        out_type=jax.ShapeDtypeStruct((NPAIR, HIDDEN), jnp.float32),
        scratch_types=[
            pltpu.VMEM((CW,), jnp.int32),
            pltpu.VMEM((CW,), jnp.int32),
            pltpu.VMEM((CW,), jnp.int32),
            pltpu.VMEM((CW,), jnp.int32),
            pltpu.VMEM((CW, HIDDEN), jnp.float32),
            pltpu.VMEM((CW, HIDDEN), jnp.float32),
            pltpu.SemaphoreType.DMA,
            pltpu.SemaphoreType.DMA,
            pltpu.SemaphoreType.DMA,
            pltpu.SemaphoreType.DMA,
        ],
    )
    def k(pos_hbm, y_hbm, y2_hbm, p0, p1, p2, p3, r0, r1,
          gs0, gs1, ss0, ss1):
        wid = lax.axis_index("s") * 2 + lax.axis_index("c")
        pbase = wid * PW
        pv = (p0, p1, p2, p3)
        for ci in range(NC):
            pltpu.sync_copy(pos_hbm.at[pl.ds(pbase + ci * CW, CW)], pv[ci])
        rb = (r0, r1)
        gs = (gs0, gs1)
        ss = (ss0, ss1)
        g = [None] * NC
        s = [None] * NC
        for ci in range(2):
            g[ci] = pltpu.async_copy(y_hbm.at[pv[ci]], rb[ci], gs[ci])
        for ci in range(NC):
            b = ci % 2
            g[ci].wait()
            s[ci] = pltpu.async_copy(
                rb[b], y2_hbm.at[pl.ds(pbase + ci * CW, CW)], ss[b])
            if ci + 2 < NC:
                s[ci].wait()
                g[ci + 2] = pltpu.async_copy(y_hbm.at[pv[ci + 2]], rb[b],
                                             gs[b])
        s[NC - 2].wait()
        s[NC - 1].wait()

    return k(ppos, y)


def _router(x, gate_w):
    return pl.pallas_call(
        _router_body,
        out_shape=[
            jax.ShapeDtypeStruct((NPAIR,), jnp.int32),
            jax.ShapeDtypeStruct((NPAIR, 1), jnp.float32),
            jax.ShapeDtypeStruct((NT1,), jnp.int32),
        ],
    )(x, gate_w)


def _wcast(w_gate, w_up, w_down):
    return pl.pallas_call(
        _wcast_body,
        grid=(E,),
        in_specs=[
            pl.BlockSpec((1, HIDDEN, FF), lambda e: (e, 0, 0)),
            pl.BlockSpec((1, HIDDEN, FF), lambda e: (e, 0, 0)),
            pl.BlockSpec((1, FF, HIDDEN), lambda e: (e, 0, 0)),
        ],
        out_specs=[
            pl.BlockSpec((1, HIDDEN, FF), lambda e: (e, 0, 0)),
            pl.BlockSpec((1, HIDDEN, FF), lambda e: (e, 0, 0)),
            pl.BlockSpec((1, FF, HIDDEN), lambda e: (e, 0, 0)),
        ],
        out_shape=[
            jax.ShapeDtypeStruct((E, HIDDEN, FF), jnp.bfloat16),
            jax.ShapeDtypeStruct((E, HIDDEN, FF), jnp.bfloat16),
            jax.ShapeDtypeStruct((E, FF, HIDDEN), jnp.bfloat16),
        ],
        compiler_params=pltpu.CompilerParams(
            dimension_semantics=("arbitrary",)
        ),
    )(w_gate, w_up, w_down)


def _ffn(te, xs, wgb, wub, wdb):
    grid_spec = pltpu.PrefetchScalarGridSpec(
        num_scalar_prefetch=1,
        grid=(NT,),
        in_specs=[
            pl.BlockSpec((BLK, HIDDEN), lambda j, s: (j, 0)),
            pl.BlockSpec((E, HIDDEN, FF), lambda j, s: (0, 0, 0)),
            pl.BlockSpec((E, HIDDEN, FF), lambda j, s: (0, 0, 0)),
            pl.BlockSpec((E, FF, HIDDEN), lambda j, s: (0, 0, 0)),
        ],
        out_specs=pl.BlockSpec((BLK, HIDDEN), lambda j, s: (j, 0)),
    )
    return pl.pallas_call(
        _ffn_body,
        grid_spec=grid_spec,
        out_shape=jax.ShapeDtypeStruct((RPAD, HIDDEN), jnp.float32),
        compiler_params=pltpu.CompilerParams(
            dimension_semantics=("arbitrary",)
        ),
    )(te, xs, wgb, wub, wdb)


def _cadd(y2, wflat):
    cb = T // 4
    nh = T // cb
    return pl.pallas_call(
        _cadd_body,
        grid=(nh,),
        in_specs=[
            pl.BlockSpec((cb, HIDDEN), lambda j: (j, 0)),
            pl.BlockSpec((cb, HIDDEN), lambda j: (j + nh, 0)),
            pl.BlockSpec((cb, 1), lambda j: (j, 0)),
            pl.BlockSpec((cb, 1), lambda j: (j + nh, 0)),
        ],
        out_specs=pl.BlockSpec((cb, HIDDEN), lambda j: (j, 0)),
        out_shape=jax.ShapeDtypeStruct((T, HIDDEN), jnp.float32),
        compiler_params=pltpu.CompilerParams(
            dimension_semantics=("parallel",)
        ),
    )(y2, y2, wflat, wflat)


@jax.jit
def kernel(hidden_states, gate_w, w_gate, w_up, w_down):
    old_shape = hidden_states.shape
    x = hidden_states.reshape(-1, old_shape[-1])

    wgb, wub, wdb = _wcast(w_gate, w_up, w_down)
    ppos, wpair, te = _router(x, gate_w)
    xs = _sc_dispatch(x, ppos)
    y = _ffn(te, xs, wgb, wub, wdb)
    y2 = _sc_combine(ppos, y)
    out = _cadd(y2, wpair)
    return out.reshape(old_shape)


# speedup vs baseline: 1.1131x; 1.1131x over previous
import functools

import jax
import jax.numpy as jnp
from jax import lax
from jax.experimental import pallas as pl
from jax.experimental.pallas import tpu as pltpu
from jax.experimental.pallas import tpu_sc as plsc

T = 2048
HIDDEN = 1024
FF = 512
E = 8
TOPK = 2
NPAIR = T * TOPK
BLK = 256
RPAD = NPAIR + E * BLK
NT = RPAD // BLK
NT1 = NT + 1
NW = 32
PW = NPAIR // NW
CW = 32
NC = PW // CW


def _router_body(x_ref, gate_ref, pos_ref, w_ref, te_ref):
    x = x_ref[...]
    gw = gate_ref[...]
    logits = lax.dot_general(
        x, gw, (((1,), (1,)), ((), ())), preferred_element_type=jnp.float32
    )
    lane = lax.broadcasted_iota(jnp.int32, (T, E), 1)
    m1 = jnp.max(logits, axis=1, keepdims=True)
    i1 = jnp.min(jnp.where(logits == m1, lane, E), axis=1, keepdims=True)
    oh1 = (lane == i1).astype(jnp.float32)
    logb = jnp.where(lane == i1, -jnp.inf, logits)
    m2 = jnp.max(logb, axis=1, keepdims=True)
    i2 = jnp.min(jnp.where(logb == m2, lane, E), axis=1, keepdims=True)
    oh2 = (lane == i2).astype(jnp.float32)
    e21 = jnp.exp(m2 - m1)
    w0 = 1.0 / (1.0 + e21)
    w1 = 1.0 - w0

    BS = 512
    r = lax.broadcasted_iota(jnp.int32, (BS, BS), 0)
    c = lax.broadcasted_iota(jnp.int32, (BS, BS), 1)
    ls = (c < r).astype(jnp.bfloat16)

    def excl_cumsum(oh):
        base = jnp.zeros((1, E), jnp.float32)
        parts = []
        for b in range(T // BS):
            sb = oh[b * BS:(b + 1) * BS]
            xb = lax.dot_general(
                ls, sb.astype(jnp.bfloat16), (((1,), (0,)), ((), ())),
                preferred_element_type=jnp.float32,
            ) + base
            parts.append(xb)
            base = base + jnp.sum(sb, axis=0, keepdims=True)
        return jnp.concatenate(parts, axis=0), base

    x1, c1 = excl_cumsum(oh1)
    x2, c2 = excl_cumsum(oh2)
    counts = c1 + c2
    padded = jnp.floor((counts + (BLK - 1)) * (1.0 / BLK)) * BLK
    acc = jnp.zeros((1, 1), jnp.float32)
    offs = []
    for e in range(E):
        offs.append(acc)
        acc = acc + padded[:, e:e + 1]
    pad_off = jnp.concatenate(offs, axis=1)
    total = acc

    rank1 = jnp.sum(x1 * oh1, axis=1, keepdims=True)
    rank2 = jnp.sum((c1 + x2) * oh2, axis=1, keepdims=True)
    off1 = jnp.sum(pad_off * oh1, axis=1, keepdims=True)
    off2 = jnp.sum(pad_off * oh2, axis=1, keepdims=True)
    pos1 = off1 + rank1
    pos2 = off2 + rank2
    pos_ref[...] = jnp.concatenate(
        [pos1, pos2], axis=0).astype(jnp.int32).reshape(NPAIR)
    w_ref[...] = jnp.concatenate([w0, w1], axis=0)

    jj = (lax.broadcasted_iota(jnp.int32, (NT1, E), 0) * BLK).astype(
        jnp.float32)
    te = jnp.sum((pad_off <= jj).astype(jnp.int32), axis=1, keepdims=True) - 1
    live = jj[:, 0:1] < total
    e8 = lax.broadcasted_iota(jnp.int32, (1, E), 1)
    last_e = jnp.max(jnp.where(padded > 0, e8, 0), axis=1, keepdims=True)
    n_live = (total * (1.0 / BLK)).astype(jnp.int32)
    tef = jnp.where(live, te, last_e)
    is_sent = lax.broadcasted_iota(jnp.int32, (NT1, 1), 0) == NT
    te_ref[...] = jnp.where(is_sent, n_live, tef).reshape(NT1)


def _wcast_body(wg_ref, wu_ref, wd_ref, wgb_ref, wub_ref, wdb_ref):
    wgb_ref[...] = wg_ref[...].astype(jnp.bfloat16)
    wub_ref[...] = wu_ref[...].astype(jnp.bfloat16)
    wdb_ref[...] = wd_ref[...].astype(jnp.bfloat16)


def _ffn_body(s_ref, xs_ref, wg_ref, wu_ref, wd_ref, y_ref):
    j = pl.program_id(0)
    e = s_ref[j]

    @pl.when(j < s_ref[NT])
    def _():
        xb = xs_ref[...]
        g = lax.dot_general(
            xb, wg_ref[0], (((1,), (0,)), ((), ())),
            preferred_element_type=jnp.float32,
        )
        u = lax.dot_general(
            xb, wu_ref[0], (((1,), (0,)), ((), ())),
            preferred_element_type=jnp.float32,
        )
        h = g * lax.logistic(g) * u
        y_ref[...] = lax.dot_general(
            h, wd_ref[0], (((1,), (0,)), ((), ())),
            preferred_element_type=jnp.float32,
        )


def _cadd_body(ya_ref, yb_ref, wa_ref, wb_ref, o_ref):
    o_ref[...] = ya_ref[...] * wa_ref[...] + yb_ref[...] * wb_ref[...]


@functools.cache
def _vector_mesh():
    return plsc.VectorSubcoreMesh(
        core_axis_name="c", subcore_axis_name="s", num_cores=2, num_subcores=16
    )


def _sc_dispatch(x, ppos):
    @functools.partial(
        pl.kernel,
        mesh=_vector_mesh(),
        out_type=jax.ShapeDtypeStruct((RPAD, HIDDEN), jnp.float32),
        scratch_types=[
            pltpu.VMEM((CW,), jnp.int32),
            pltpu.VMEM((CW,), jnp.int32),
            pltpu.VMEM((CW,), jnp.int32),
            pltpu.VMEM((CW,), jnp.int32),
            pltpu.VMEM((CW, HIDDEN), jnp.float32),
            pltpu.VMEM((CW, HIDDEN), jnp.float32),
            pltpu.SemaphoreType.DMA,
            pltpu.SemaphoreType.DMA,
            pltpu.SemaphoreType.DMA,
            pltpu.SemaphoreType.DMA,
        ],
    )
    def k(x_hbm, pos_hbm, xs_hbm, p0, p1, p2, p3, r0, r1,
          gs0, gs1, ss0, ss1):
        wid = lax.axis_index("s") * 2 + lax.axis_index("c")
        pbase = wid * PW
        tbase = pbase % T
        pv = (p0, p1, p2, p3)
        for ci in range(NC):
            pltpu.sync_copy(pos_hbm.at[pl.ds(pbase + ci * CW, CW)], pv[ci])
        rb = (r0, r1)
        gs = (gs0, gs1)
        ss = (ss0, ss1)
        g = [None] * NC
        s = [None] * NC
        for ci in range(2):
            g[ci] = pltpu.async_copy(
                x_hbm.at[pl.ds(tbase + ci * CW, CW)], rb[ci], gs[ci])
        for ci in range(NC):
            b = ci % 2
            g[ci].wait()
            s[ci] = pltpu.async_copy(rb[b], xs_hbm.at[pv[ci]], ss[b])
            if ci + 2 < NC:
                s[ci].wait()
                g[ci + 2] = pltpu.async_copy(
                    x_hbm.at[pl.ds(tbase + (ci + 2) * CW, CW)], rb[b], gs[b])
        s[NC - 2].wait()
        s[NC - 1].wait()

    return k(x, ppos)


def _sc_combine(ppos, y):
    @functools.partial(
        pl.kernel,
        mesh=_vector_mesh(),
        out_type=jax.ShapeDtypeStruct((NPAIR, HIDDEN), jnp.float32),
        scratch_types=[
            pltpu.VMEM((CW,), jnp.int32),
            pltpu.VMEM((CW,), jnp.int32),
            pltpu.VMEM((CW,), jnp.int32),
            pltpu.VMEM((CW,), jnp.int32),
            pltpu.VMEM((CW, HIDDEN), jnp.float32),
            pltpu.VMEM((CW, HIDDEN), jnp.float32),
            pltpu.SemaphoreType.DMA,
            pltpu.SemaphoreType.DMA,
            pltpu.SemaphoreType.DMA,
            pltpu.SemaphoreType.DMA,
        ],
    )
    def k(pos_hbm, y_hbm, y2_hbm, p0, p1, p2, p3, r0, r1,
          gs0, gs1, ss0, ss1):
        wid = lax.axis_index("s") * 2 + lax.axis_index("c")
        pbase = wid * PW
        pv = (p0, p1, p2, p3)
        for ci in range(NC):
            pltpu.sync_copy(pos_hbm.at[pl.ds(pbase + ci * CW, CW)], pv[ci])
        rb = (r0, r1)
        gs = (gs0, gs1)
        ss = (ss0, ss1)
        g = [None] * NC
        s = [None] * NC
        for ci in range(2):
            g[ci] = pltpu.async_copy(y_hbm.at[pv[ci]], rb[ci], gs[ci])
        for ci in range(NC):
            b = ci % 2
            g[ci].wait()
            s[ci] = pltpu.async_copy(
                rb[b], y2_hbm.at[pl.ds(pbase + ci * CW, CW)], ss[b])
            if ci + 2 < NC:
                s[ci].wait()
                g[ci + 2] = pltpu.async_copy(y_hbm.at[pv[ci + 2]], rb[b],
                                             gs[b])
        s[NC - 2].wait()
        s[NC - 1].wait()

    return k(ppos, y)


def _router(x, gate_w):
    return pl.pallas_call(
        _router_body,
        out_shape=[
            jax.ShapeDtypeStruct((NPAIR,), jnp.int32),
            jax.ShapeDtypeStruct((NPAIR, 1), jnp.float32),
            jax.ShapeDtypeStruct((NT1,), jnp.int32),
        ],
    )(x, gate_w)


def _wcast(w_gate, w_up, w_down):
    return pl.pallas_call(
        _wcast_body,
        grid=(E,),
        in_specs=[
            pl.BlockSpec((1, HIDDEN, FF), lambda e: (e, 0, 0)),
            pl.BlockSpec((1, HIDDEN, FF), lambda e: (e, 0, 0)),
            pl.BlockSpec((1, FF, HIDDEN), lambda e: (e, 0, 0)),
        ],
        out_specs=[
            pl.BlockSpec((1, HIDDEN, FF), lambda e: (e, 0, 0)),
            pl.BlockSpec((1, HIDDEN, FF), lambda e: (e, 0, 0)),
            pl.BlockSpec((1, FF, HIDDEN), lambda e: (e, 0, 0)),
        ],
        out_shape=[
            jax.ShapeDtypeStruct((E, HIDDEN, FF), jnp.bfloat16),
            jax.ShapeDtypeStruct((E, HIDDEN, FF), jnp.bfloat16),
            jax.ShapeDtypeStruct((E, FF, HIDDEN), jnp.bfloat16),
        ],
        compiler_params=pltpu.CompilerParams(
            dimension_semantics=("arbitrary",)
        ),
    )(w_gate, w_up, w_down)


def _ffn(te, xs, wgb, wub, wdb):
    grid_spec = pltpu.PrefetchScalarGridSpec(
        num_scalar_prefetch=1,
        grid=(NT,),
        in_specs=[
            pl.BlockSpec((BLK, HIDDEN),
                         lambda j, s: (jnp.minimum(j, s[NT] - 1), 0)),
            pl.BlockSpec((1, HIDDEN, FF), lambda j, s: (s[j], 0, 0)),
            pl.BlockSpec((1, HIDDEN, FF), lambda j, s: (s[j], 0, 0)),
            pl.BlockSpec((1, FF, HIDDEN), lambda j, s: (s[j], 0, 0)),
        ],
        out_specs=pl.BlockSpec(
            (BLK, HIDDEN), lambda j, s: (jnp.minimum(j, s[NT] - 1), 0)),
    )
    return pl.pallas_call(
        _ffn_body,
        grid_spec=grid_spec,
        out_shape=jax.ShapeDtypeStruct((RPAD, HIDDEN), jnp.float32),
        compiler_params=pltpu.CompilerParams(
            dimension_semantics=("arbitrary",)
        ),
    )(te, xs, wgb, wub, wdb)


def _cadd(y2, wflat):
    cb = T // 4
    nh = T // cb
    return pl.pallas_call(
        _cadd_body,
        grid=(nh,),
        in_specs=[
            pl.BlockSpec((cb, HIDDEN), lambda j: (j, 0)),
            pl.BlockSpec((cb, HIDDEN), lambda j: (j + nh, 0)),
            pl.BlockSpec((cb, 1), lambda j: (j, 0)),
            pl.BlockSpec((cb, 1), lambda j: (j + nh, 0)),
        ],
        out_specs=pl.BlockSpec((cb, HIDDEN), lambda j: (j, 0)),
        out_shape=jax.ShapeDtypeStruct((T, HIDDEN), jnp.float32),
        compiler_params=pltpu.CompilerParams(
            dimension_semantics=("parallel",)
        ),
    )(y2, y2, wflat, wflat)


@jax.jit
def kernel(hidden_states, gate_w, w_gate, w_up, w_down):
    old_shape = hidden_states.shape
    x = hidden_states.reshape(-1, old_shape[-1])

    ppos, wpair, te = _router(x, gate_w)
    xs = _sc_dispatch(x, ppos)
    y = _ffn(te, xs, w_gate, w_up, w_down)
    y2 = _sc_combine(ppos, y)
    out = _cadd(y2, wpair)
    return out.reshape(old_shape)
